# Initial kernel scaffold; baseline (speedup 1.0000x reference)
import jax
import jax.numpy as jnp
from jax.experimental import pallas as pl


def kernel(xyz, feat, W1, b1, g1, be1, Wg, bg, gg, beg, Wl, bl, gl, bel, W2, b2, g2, be2, W3, b3, g3, be3):
    raise NotImplementedError("write your pallas kernel here")


# speedup vs baseline: 1.7398x; 1.7398x over previous
import functools
import jax
import jax.numpy as jnp
from jax.experimental import pallas as pl
from jax.experimental.pallas import tpu as pltpu

B, N, CIN = 8, 2048, 128
RED = CIN // 2
KG = 8
KL = 16
EPS = 1e-5

ROWS = 256
NEG_BIG = 3.4e38


def _topk_body(xr_ref, xc_ref, idx_ref):
    xr = xr_ref[0]
    xc = xc_ref[0]
    inner = 2.0 * jax.lax.dot_general(
        xr, xc, (((1,), (0,)), ((), ())),
        preferred_element_type=jnp.float32)
    x0r = xr[:, 0:1]
    x1r = xr[:, 1:2]
    x0c = xc[0:1, :]
    x1c = xc[1:2, :]
    xxr = x0r * x0r + x1r * x1r
    xxc = x0c * x0c + x1c * x1c
    s = -((xxc - inner) + xxr)
    col = jax.lax.broadcasted_iota(jnp.int32, (ROWS, N), 1)
    cols = []
    for _ in range(KL):
        m = jnp.max(s, axis=1, keepdims=True)
        hit = s == m
        ind = jnp.min(jnp.where(hit, col, N), axis=1, keepdims=True)
        cols.append(ind)
        s = jnp.where(col == ind, jnp.float32(-NEG_BIG), s)
    idx_ref[0] = jnp.concatenate(cols, axis=1)


def _topk16(xyz):
    xc = xyz[:, 0:2, :]
    xr = jnp.transpose(xc, (0, 2, 1))
    grid = (B, N // ROWS)
    return pl.pallas_call(
        _topk_body,
        grid=grid,
        in_specs=[
            pl.BlockSpec((1, ROWS, 2), lambda b, r: (b, r, 0)),
            pl.BlockSpec((1, 2, N), lambda b, r: (b, 0, 0)),
        ],
        out_specs=pl.BlockSpec((1, ROWS, KL), lambda b, r: (b, r, 0)),
        out_shape=jax.ShapeDtypeStruct((B, N, KL), jnp.int32),
    )(xr, xc)


def _bn(x, g, b):
    m = jnp.mean(x, axis=(0, 2), keepdims=True)
    v = jnp.var(x, axis=(0, 2), keepdims=True)
    return g[None, :, None] * (x - m) / jnp.sqrt(v + EPS) + b[None, :, None]


def _conv1(x, W, b):
    return jnp.einsum('oc,bcn->bon', W, x) + b[None, :, None]


def kernel(xyz, feat, W1, b1, g1, be1, Wg, bg, gg, beg, Wl, bl, gl, bel,
           W2, b2, g2, be2, W3, b3, g3, be3):
    idx16 = _topk16(xyz)
    idx = idx16[:, :, :KG]

    x = jax.nn.relu(_bn(_conv1(feat, W1, b1), g1, be1))

    def gath(fb, ib):
        return fb[:, ib.reshape(-1)].reshape(RED, KG, N)
    group = jax.vmap(gath)(x, idx)
    group = jnp.transpose(group, (0, 3, 2, 1))
    h = jnp.einsum('bnkf,of->bnko', group, Wg) + bg
    h = h.reshape(B, N * KG, RED)
    h = jax.nn.relu(_bn(jnp.transpose(h, (0, 2, 1)), gg, beg))
    x = jnp.max(h.reshape(B, RED, N, KG), axis=-1)

    def lap(fb, ib):
        lf = fb[:, ib.reshape(-1)].reshape(N, KL, RED)
        return fb.T - jnp.mean(lf, axis=1)
    lapfeat = jnp.transpose(jax.vmap(lap)(x, idx16), (0, 2, 1))
    t = jnp.einsum('bfn,of->bon', lapfeat, Wl) + bl[None, :, None]
    t = jax.nn.relu(_bn(t, gl, bel))
    x = x + t
    y = jax.nn.relu(_bn(_conv1(x, W2, b2), g2, be2)) + feat
    return jax.nn.relu(_bn(_conv1(y, W3, b3), g3, be3))


# speedup vs baseline: 9.1631x; 5.2669x over previous
import functools
import jax
import jax.numpy as jnp
from jax import lax
from jax.experimental import pallas as pl
from jax.experimental.pallas import tpu as pltpu
from jax.experimental.pallas import tpu_sc as plsc

B, N, CIN = 8, 2048, 128
RED = CIN // 2
KG = 8
KL = 16
EPS = 1e-5

ROWS = 256
NEG_BIG = 3.4e38

_ARB3 = pltpu.CompilerParams(dimension_semantics=("arbitrary",) * 3)
_SC_PARAMS = pltpu.CompilerParams(needs_layout_passes=False)


def _topk_body(xr_ref, xc_ref, idx_ref):
    xr = xr_ref[0]
    xc = xc_ref[0]
    inner = 2.0 * jax.lax.dot_general(
        xr, xc, (((1,), (0,)), ((), ())),
        preferred_element_type=jnp.float32)
    x0r = xr[:, 0:1]
    x1r = xr[:, 1:2]
    x0c = xc[0:1, :]
    x1c = xc[1:2, :]
    xxr = x0r * x0r + x1r * x1r
    xxc = x0c * x0c + x1c * x1c
    s = -((xxc - inner) + xxr)
    col = jax.lax.broadcasted_iota(jnp.int32, (ROWS, N), 1)
    cols = []
    for _ in range(KL):
        m = jnp.max(s, axis=1, keepdims=True)
        hit = s == m
        ind = jnp.min(jnp.where(hit, col, N), axis=1, keepdims=True)
        cols.append(ind)
        s = jnp.where(col == ind, jnp.float32(-NEG_BIG), s)
    idx_ref[0] = jnp.concatenate(cols, axis=1)


def _topk16(xyz):
    xc = xyz[:, 0:2, :]
    xr = jnp.transpose(xc, (0, 2, 1))
    return pl.pallas_call(
        _topk_body,
        grid=(B, N // ROWS),
        in_specs=[
            pl.BlockSpec((1, ROWS, 2), lambda b, r: (b, r, 0)),
            pl.BlockSpec((1, 2, N), lambda b, r: (b, 0, 0)),
        ],
        out_specs=pl.BlockSpec((1, ROWS, KL), lambda b, r: (b, r, 0)),
        out_shape=jax.ShapeDtypeStruct((B, N, KL), jnp.int32),
    )(xr, xc)


NB1 = 1024
CNT1 = float(B * N)


def _k1_body(featT_ref, w1t_ref, g_ref, be_ref, out_ref, st_ref):
    p = pl.program_id(0)
    b = pl.program_id(1)
    nb = pl.program_id(2)
    z = jax.lax.dot_general(
        featT_ref[0], w1t_ref[...], (((1,), (0,)), ((), ())),
        preferred_element_type=jnp.float32)

    @pl.when((p == 0) & (b == 0) & (nb == 0))
    def _():
        st_ref[...] = jnp.zeros_like(st_ref)

    @pl.when(p == 0)
    def _():
        st_ref[0:1, :] += jnp.sum(z, axis=0, keepdims=True)
        st_ref[1:2, :] += jnp.sum(z * z, axis=0, keepdims=True)

    @pl.when(p == 1)
    def _():
        mean = st_ref[0:1, :] / CNT1
        var = st_ref[1:2, :] / CNT1 - mean * mean
        scale = g_ref[...] * jax.lax.rsqrt(var + EPS)
        shift = be_ref[...] - mean * scale
        zn = jnp.maximum(z * scale + shift, 0.0)
        out_ref[0] = jnp.concatenate(
            [zn, jnp.zeros((NB1, CIN - RED), jnp.float32)], axis=1)


def _stage1(featT, W1, g1, be1):
    return pl.pallas_call(
        _k1_body,
        grid=(2, B, N // NB1),
        in_specs=[
            pl.BlockSpec((1, NB1, CIN), lambda p, b, nb: (b, nb, 0)),
            pl.BlockSpec((CIN, RED), lambda p, b, nb: (0, 0)),
            pl.BlockSpec((1, RED), lambda p, b, nb: (0, 0)),
            pl.BlockSpec((1, RED), lambda p, b, nb: (0, 0)),
        ],
        out_specs=pl.BlockSpec((1, NB1, CIN), lambda p, b, nb: (b, nb, 0)),
        out_shape=jax.ShapeDtypeStruct((B, N, CIN), jnp.float32),
        scratch_shapes=[pltpu.VMEM((8, RED), jnp.float32)],
        compiler_params=_ARB3,
    )(featT, W1.T, g1.reshape(1, RED), be1.reshape(1, RED))


NW = 32
GROWS = (B * N * KG) // NW
GCH = 512


def _sc_group_gather(table, jg2):
    mesh = plsc.VectorSubcoreMesh(core_axis_name="c", subcore_axis_name="s")

    @functools.partial(
        pl.kernel, mesh=mesh,
        out_type=jax.ShapeDtypeStruct((B * N * KG, CIN), jnp.float32),
        compiler_params=_SC_PARAMS,
        scratch_types=[
            pltpu.VMEM((GROWS // 128, 128), jnp.int32),
            pltpu.VMEM((GCH, CIN), jnp.float32),
            pltpu.SemaphoreType.DMA,
        ],
    )
    def k(table_hbm, jg_hbm, out_hbm, idx_v, rows_v, sem):
        wid = lax.axis_index("s") * 2 + lax.axis_index("c")
        base = wid * GROWS
        pltpu.sync_copy(
            jg_hbm.at[pl.ds(pl.multiple_of(wid * (GROWS // 128),
                                           GROWS // 128), GROWS // 128)],
            idx_v)
        for c in range(GROWS // GCH):
            off = base + c * GCH
            for i in range(GCH // 128):
                pltpu.async_copy(
                    table_hbm.at[idx_v.at[c * (GCH // 128) + i]],
                    rows_v.at[pl.ds(i * 128, 128)], sem).wait()
            pltpu.sync_copy(rows_v, out_hbm.at[pl.ds(off, GCH)])

    return k(table, jg2)


MB3 = 2048
NP3 = MB3 // KG
CNT3 = float(B * N * KG)


def _k3_body(g_ref, wgt_ref, bg_ref, gg_ref, beg_ref, pm_ref, cm_ref, st_ref):
    p = pl.program_id(0)
    b = pl.program_id(1)
    mb = pl.program_id(2)
    h = jax.lax.dot_general(
        g_ref[:, :RED], wgt_ref[...], (((1,), (0,)), ((), ())),
        preferred_element_type=jnp.float32) + bg_ref[...]

    @pl.when((p == 0) & (b == 0) & (mb == 0))
    def _():
        st_ref[...] = jnp.zeros_like(st_ref)

    @pl.when(p == 0)
    def _():
        st_ref[0:1, :] += jnp.sum(h, axis=0, keepdims=True)
        st_ref[1:2, :] += jnp.sum(h * h, axis=0, keepdims=True)

    @pl.when(p == 1)
    def _():
        mean = st_ref[0:1, :] / CNT3
        var = st_ref[1:2, :] / CNT3 - mean * mean
        scale = gg_ref[...] * jax.lax.rsqrt(var + EPS)
        shift = beg_ref[...] - mean * scale
        hn = jnp.maximum(h * scale + shift, 0.0)
        x2 = jnp.max(hn.reshape(NP3, KG, RED), axis=1)
        pm_ref[0] = x2
        cm_ref[0] = x2.T


def _stage3(G, Wg, bg, gg, beg):
    return pl.pallas_call(
        _k3_body,
        grid=(2, B, (N * KG) // MB3),
        in_specs=[
            pl.BlockSpec((MB3, CIN), lambda p, b, mb: (b * (N * KG // MB3) + mb, 0)),
            pl.BlockSpec((RED, RED), lambda p, b, mb: (0, 0)),
            pl.BlockSpec((1, RED), lambda p, b, mb: (0, 0)),
            pl.BlockSpec((1, RED), lambda p, b, mb: (0, 0)),
            pl.BlockSpec((1, RED), lambda p, b, mb: (0, 0)),
        ],
        out_specs=[
            pl.BlockSpec((1, NP3, RED), lambda p, b, mb: (b, mb, 0)),
            pl.BlockSpec((1, RED, NP3), lambda p, b, mb: (b, 0, mb)),
        ],
        out_shape=[
            jax.ShapeDtypeStruct((B, N, RED), jnp.float32),
            jax.ShapeDtypeStruct((B, RED, N), jnp.float32),
        ],
        scratch_shapes=[pltpu.VMEM((8, RED), jnp.float32)],
        compiler_params=_ARB3,
    )(G, Wg.T, bg.reshape(1, RED), gg.reshape(1, RED), beg.reshape(1, RED))


QPW = 16


def _sc_lap(x2cm, idxf):
    mesh = plsc.VectorSubcoreMesh(core_axis_name="c", subcore_axis_name="s")

    @functools.partial(
        pl.kernel, mesh=mesh,
        out_type=jax.ShapeDtypeStruct((B * N * RED,), jnp.float32),
        compiler_params=_SC_PARAMS,
        scratch_types=[
            pltpu.VMEM((N * KL,), jnp.int32),
            pltpu.VMEM((QPW * N,), jnp.float32),
            pltpu.VMEM((QPW * 128 * 16,), jnp.float32),
            pltpu.SemaphoreType.DMA,
        ],
    )
    def k(x2_hbm, idx_hbm, out_hbm, idx_v, fb_v, acc_v, sem):
        wid = lax.axis_index("s") * 2 + lax.axis_index("c")
        b = wid // 4
        qbase = (wid % 4) * QPW
        pltpu.sync_copy(idx_hbm.at[b], idx_v)
        for ql in range(QPW):
            pltpu.sync_copy(x2_hbm.at[b, qbase + ql],
                            fb_v.at[pl.ds(ql * N, N)])

        def zero(i, _):
            acc_v[pl.ds(i * 16, 16)] = jnp.zeros((16,), jnp.float32)
            return 0
        lax.fori_loop(0, QPW * 128, zero, 0, unroll=4)

        for ql in range(QPW):
            fboff = ql * N
            arow = ql * 128

            def body(m, _):
                iv = idx_v[pl.ds(m * 16, 16)]
                v = plsc.load_gather(fb_v, [iv + fboff])
                a16 = (arow + (m >> 6) * 4 + (m & 3)) * 16
                acc_v[pl.ds(a16, 16)] += v
                return 0
            lax.fori_loop(0, N, body, 0, unroll=4)

        out0 = (b * N + 32 * qbase) * RED
        pltpu.sync_copy(acc_v, out_hbm.at[pl.ds(out0, QPW * 128 * 16)])

    return k(x2cm, idxf)


NB5 = 1024
CNT5 = float(B * N)


def _k5_body(x2_ref, mn_ref, wlt_ref, bl_ref, g_ref, be_ref, out_ref, st_ref):
    p = pl.program_id(0)
    b = pl.program_id(1)
    nb = pl.program_id(2)
    x2 = x2_ref[0]
    lap = x2 - mn_ref[0] * (1.0 / KL)
    traw = jax.lax.dot_general(
        lap, wlt_ref[...], (((1,), (0,)), ((), ())),
        preferred_element_type=jnp.float32) + bl_ref[...]

    @pl.when((p == 0) & (b == 0) & (nb == 0))
    def _():
        st_ref[...] = jnp.zeros_like(st_ref)

    @pl.when(p == 0)
    def _():
        st_ref[0:1, :] += jnp.sum(traw, axis=0, keepdims=True)
        st_ref[1:2, :] += jnp.sum(traw * traw, axis=0, keepdims=True)

    @pl.when(p == 1)
    def _():
        mean = st_ref[0:1, :] / CNT5
        var = st_ref[1:2, :] / CNT5 - mean * mean
        scale = g_ref[...] * jax.lax.rsqrt(var + EPS)
        shift = be_ref[...] - mean * scale
        x3 = x2 + jnp.maximum(traw * scale + shift, 0.0)
        out_ref[0] = x3.T


def _stage5(x2pm, meanS, Wl, bl, gl, bel):
    return pl.pallas_call(
        _k5_body,
        grid=(2, B, N // NB5),
        in_specs=[
            pl.BlockSpec((1, NB5, RED), lambda p, b, nb: (b, nb, 0)),
            pl.BlockSpec((1, NB5, RED), lambda p, b, nb: (b, nb, 0)),
            pl.BlockSpec((RED, RED), lambda p, b, nb: (0, 0)),
            pl.BlockSpec((1, RED), lambda p, b, nb: (0, 0)),
            pl.BlockSpec((1, RED), lambda p, b, nb: (0, 0)),
            pl.BlockSpec((1, RED), lambda p, b, nb: (0, 0)),
        ],
        out_specs=pl.BlockSpec((1, RED, NB5), lambda p, b, nb: (b, 0, nb)),
        out_shape=jax.ShapeDtypeStruct((B, RED, N), jnp.float32),
        scratch_shapes=[pltpu.VMEM((8, RED), jnp.float32)],
        compiler_params=_ARB3,
    )(x2pm, meanS, Wl.T, bl.reshape(1, RED), gl.reshape(1, RED),
      bel.reshape(1, RED))


NB6 = 1024
CNT6 = float(B * N)


def _k6_body(x3_ref, feat_ref, w2_ref, b2_ref, g2_ref, be2_ref,
             w3_ref, b3_ref, g3_ref, be3_ref, out_ref, st2_ref, st3_ref):
    p = pl.program_id(0)
    b = pl.program_id(1)
    nb = pl.program_id(2)
    yraw = jax.lax.dot_general(
        w2_ref[...], x3_ref[0], (((1,), (0,)), ((), ())),
        preferred_element_type=jnp.float32) + b2_ref[...]

    @pl.when((p == 0) & (b == 0) & (nb == 0))
    def _():
        st2_ref[...] = jnp.zeros_like(st2_ref)
        st3_ref[...] = jnp.zeros_like(st3_ref)

    @pl.when(p == 0)
    def _():
        st2_ref[:, 0:1] += jnp.sum(yraw, axis=1, keepdims=True)
        st2_ref[:, 1:2] += jnp.sum(yraw * yraw, axis=1, keepdims=True)

    @pl.when(p >= 1)
    def _():
        mean2 = st2_ref[:, 0:1] / CNT6
        var2 = st2_ref[:, 1:2] / CNT6 - mean2 * mean2
        scale2 = g2_ref[...] * jax.lax.rsqrt(var2 + EPS)
        shift2 = be2_ref[...] - mean2 * scale2
        y = jnp.maximum(yraw * scale2 + shift2, 0.0) + feat_ref[0]
        zraw = jax.lax.dot_general(
            w3_ref[...], y, (((1,), (0,)), ((), ())),
            preferred_element_type=jnp.float32) + b3_ref[...]

        @pl.when(p == 1)
        def _():
            st3_ref[:, 0:1] += jnp.sum(zraw, axis=1, keepdims=True)
            st3_ref[:, 1:2] += jnp.sum(zraw * zraw, axis=1, keepdims=True)

        @pl.when(p == 2)
        def _():
            mean3 = st3_ref[:, 0:1] / CNT6
            var3 = st3_ref[:, 1:2] / CNT6 - mean3 * mean3
            scale3 = g3_ref[...] * jax.lax.rsqrt(var3 + EPS)
            shift3 = be3_ref[...] - mean3 * scale3
            out_ref[0] = jnp.maximum(zraw * scale3 + shift3, 0.0)


def _stage6(x3cm, feat, W2, b2, g2, be2, W3, b3, g3, be3):
    C2 = 2 * CIN
    return pl.pallas_call(
        _k6_body,
        grid=(3, B, N // NB6),
        in_specs=[
            pl.BlockSpec((1, RED, NB6), lambda p, b, nb: (b, 0, nb)),
            pl.BlockSpec((1, CIN, NB6), lambda p, b, nb: (b, 0, nb)),
            pl.BlockSpec((CIN, RED), lambda p, b, nb: (0, 0)),
            pl.BlockSpec((CIN, 1), lambda p, b, nb: (0, 0)),
            pl.BlockSpec((CIN, 1), lambda p, b, nb: (0, 0)),
            pl.BlockSpec((CIN, 1), lambda p, b, nb: (0, 0)),
            pl.BlockSpec((C2, CIN), lambda p, b, nb: (0, 0)),
            pl.BlockSpec((C2, 1), lambda p, b, nb: (0, 0)),
            pl.BlockSpec((C2, 1), lambda p, b, nb: (0, 0)),
            pl.BlockSpec((C2, 1), lambda p, b, nb: (0, 0)),
        ],
        out_specs=pl.BlockSpec((1, C2, NB6), lambda p, b, nb: (b, 0, nb)),
        out_shape=jax.ShapeDtypeStruct((B, C2, N), jnp.float32),
        scratch_shapes=[
            pltpu.VMEM((CIN, 8), jnp.float32),
            pltpu.VMEM((C2, 8), jnp.float32),
        ],
        compiler_params=_ARB3,
    )(x3cm, feat, W2, b2.reshape(CIN, 1), g2.reshape(CIN, 1),
      be2.reshape(CIN, 1), W3, b3.reshape(C2, 1), g3.reshape(C2, 1),
      be3.reshape(C2, 1))


def kernel(xyz, feat, W1, b1, g1, be1, Wg, bg, gg, beg, Wl, bl, gl, bel,
           W2, b2, g2, be2, W3, b3, g3, be3):
    idx16 = _topk16(xyz)

    idx8 = idx16[:, :, :KG]
    Jg = jnp.transpose(idx8.reshape(B, KG, N // KG, KG), (0, 2, 3, 1))
    Jg = Jg.reshape(B, N * KG) + (jnp.arange(B, dtype=jnp.int32) * N)[:, None]
    Jg = Jg.reshape(B * N * KG // 128, 128)

    featT = jnp.transpose(feat, (0, 2, 1))
    z1 = _stage1(featT, W1, g1, be1)

    G = _sc_group_gather(z1.reshape(B * N, CIN), Jg)
    x2pm, x2cm = _stage3(G, Wg, bg, gg, beg)

    accf = _sc_lap(x2cm, idx16.reshape(B, N * KL))
    meanS = accf.reshape(B, N, RED)

    x3cm = _stage5(x2pm, meanS, Wl, bl, gl, bel)
    return _stage6(x3cm, feat, W2, b2, g2, be2, W3, b3, g3, be3)


# speedup vs baseline: 11.6566x; 1.2721x over previous
import functools
import jax
import jax.numpy as jnp
from jax import lax
from jax.experimental import pallas as pl
from jax.experimental.pallas import tpu as pltpu
from jax.experimental.pallas import tpu_sc as plsc

B, N, CIN = 8, 2048, 128
RED = CIN // 2
KG = 8
KL = 16
EPS = 1e-5

ROWS = 256
NEG_BIG = 3.4e38

_ARB3 = pltpu.CompilerParams(dimension_semantics=("arbitrary",) * 3)
_SC_PARAMS = pltpu.CompilerParams(needs_layout_passes=False)


def _topk_body(xr_ref, xc_ref, idx_ref):
    xr = xr_ref[0]
    xc = xc_ref[0]
    inner = 2.0 * jax.lax.dot_general(
        xr, xc, (((1,), (0,)), ((), ())),
        preferred_element_type=jnp.float32)
    x0r = xr[:, 0:1]
    x1r = xr[:, 1:2]
    x0c = xc[0:1, :]
    x1c = xc[1:2, :]
    xxr = x0r * x0r + x1r * x1r
    xxc = x0c * x0c + x1c * x1c
    s = -((xxc - inner) + xxr)
    col = jax.lax.broadcasted_iota(jnp.int32, (ROWS, N), 1)
    cols = []
    for _ in range(KL):
        m = jnp.max(s, axis=1, keepdims=True)
        hit = s == m
        ind = jnp.min(jnp.where(hit, col, N), axis=1, keepdims=True)
        cols.append(ind)
        s = jnp.where(col == ind, jnp.float32(-NEG_BIG), s)
    idx_ref[0] = jnp.concatenate(cols, axis=1)


def _topk16(xyz):
    xc = xyz[:, 0:2, :]
    xr = jnp.transpose(xc, (0, 2, 1))
    return pl.pallas_call(
        _topk_body,
        grid=(B, N // ROWS),
        in_specs=[
            pl.BlockSpec((1, ROWS, 2), lambda b, r: (b, r, 0)),
            pl.BlockSpec((1, 2, N), lambda b, r: (b, 0, 0)),
        ],
        out_specs=pl.BlockSpec((1, ROWS, KL), lambda b, r: (b, r, 0)),
        out_shape=jax.ShapeDtypeStruct((B, N, KL), jnp.int32),
    )(xr, xc)


NB1 = 1024
CNT1 = float(B * N)


def _k1_body(featT_ref, w1t_ref, g_ref, be_ref, out_ref, st_ref):
    p = pl.program_id(0)
    b = pl.program_id(1)
    nb = pl.program_id(2)
    z = jax.lax.dot_general(
        featT_ref[0], w1t_ref[...], (((1,), (0,)), ((), ())),
        preferred_element_type=jnp.float32)

    @pl.when((p == 0) & (b == 0) & (nb == 0))
    def _():
        st_ref[...] = jnp.zeros_like(st_ref)

    @pl.when(p == 0)
    def _():
        st_ref[0:1, :] += jnp.sum(z, axis=0, keepdims=True)
        st_ref[1:2, :] += jnp.sum(z * z, axis=0, keepdims=True)

    @pl.when(p == 1)
    def _():
        mean = st_ref[0:1, :] / CNT1
        var = st_ref[1:2, :] / CNT1 - mean * mean
        scale = g_ref[...] * jax.lax.rsqrt(var + EPS)
        shift = be_ref[...] - mean * scale
        zn = jnp.maximum(z * scale + shift, 0.0)
        out_ref[0] = jnp.concatenate(
            [zn, jnp.zeros((NB1, CIN - RED), jnp.float32)], axis=1)


def _stage1(featT, W1, g1, be1):
    return pl.pallas_call(
        _k1_body,
        grid=(2, B, N // NB1),
        in_specs=[
            pl.BlockSpec((1, NB1, CIN), lambda p, b, nb: (b, nb, 0)),
            pl.BlockSpec((CIN, RED), lambda p, b, nb: (0, 0)),
            pl.BlockSpec((1, RED), lambda p, b, nb: (0, 0)),
            pl.BlockSpec((1, RED), lambda p, b, nb: (0, 0)),
        ],
        out_specs=pl.BlockSpec((1, NB1, CIN), lambda p, b, nb: (b, nb, 0)),
        out_shape=jax.ShapeDtypeStruct((B, N, CIN), jnp.float32),
        scratch_shapes=[pltpu.VMEM((8, RED), jnp.float32)],
        compiler_params=_ARB3,
    )(featT, W1.T, g1.reshape(1, RED), be1.reshape(1, RED))


NW = 32
GROWS = (B * N * KG) // NW
GCH = 512


def _sc_group_gather(table, jg2):
    mesh = plsc.VectorSubcoreMesh(core_axis_name="c", subcore_axis_name="s")

    @functools.partial(
        pl.kernel, mesh=mesh,
        out_type=jax.ShapeDtypeStruct((B * N * KG, CIN), jnp.float32),
        compiler_params=_SC_PARAMS,
        scratch_types=[
            pltpu.VMEM((GROWS // 128, 128), jnp.int32),
            pltpu.VMEM((GCH, CIN), jnp.float32),
            pltpu.SemaphoreType.DMA,
        ],
    )
    def k(table_hbm, jg_hbm, out_hbm, idx_v, rows_v, sem):
        wid = lax.axis_index("s") * 2 + lax.axis_index("c")
        base = wid * GROWS
        pltpu.sync_copy(
            jg_hbm.at[pl.ds(pl.multiple_of(wid * (GROWS // 128),
                                           GROWS // 128), GROWS // 128)],
            idx_v)
        for c in range(GROWS // GCH):
            off = base + c * GCH
            for i in range(GCH // 128):
                pltpu.async_copy(
                    table_hbm.at[idx_v.at[c * (GCH // 128) + i]],
                    rows_v.at[pl.ds(i * 128, 128)], sem).wait()
            pltpu.sync_copy(rows_v, out_hbm.at[pl.ds(off, GCH)])

    return k(table, jg2)


MB3 = 2048
NP3 = MB3 // KG
CNT3 = float(B * N * KG)


def _k3_body(g_ref, wgt_ref, bg_ref, gg_ref, beg_ref, pm_ref, cm_ref, st_ref):
    p = pl.program_id(0)
    b = pl.program_id(1)
    mb = pl.program_id(2)
    h = jax.lax.dot_general(
        g_ref[:, :RED], wgt_ref[...], (((1,), (0,)), ((), ())),
        preferred_element_type=jnp.float32) + bg_ref[...]

    @pl.when((p == 0) & (b == 0) & (mb == 0))
    def _():
        st_ref[...] = jnp.zeros_like(st_ref)

    @pl.when(p == 0)
    def _():
        st_ref[0:1, :] += jnp.sum(h, axis=0, keepdims=True)
        st_ref[1:2, :] += jnp.sum(h * h, axis=0, keepdims=True)

    @pl.when(p == 1)
    def _():
        mean = st_ref[0:1, :] / CNT3
        var = st_ref[1:2, :] / CNT3 - mean * mean
        scale = gg_ref[...] * jax.lax.rsqrt(var + EPS)
        shift = beg_ref[...] - mean * scale
        hn = jnp.maximum(h * scale + shift, 0.0)
        x2 = jnp.max(hn.reshape(NP3, KG, RED), axis=1)
        pm_ref[0] = x2
        cm_ref[0] = x2.T


def _stage3(G, Wg, bg, gg, beg):
    return pl.pallas_call(
        _k3_body,
        grid=(2, B, (N * KG) // MB3),
        in_specs=[
            pl.BlockSpec((MB3, CIN), lambda p, b, mb: (b * (N * KG // MB3) + mb, 0)),
            pl.BlockSpec((RED, RED), lambda p, b, mb: (0, 0)),
            pl.BlockSpec((1, RED), lambda p, b, mb: (0, 0)),
            pl.BlockSpec((1, RED), lambda p, b, mb: (0, 0)),
            pl.BlockSpec((1, RED), lambda p, b, mb: (0, 0)),
        ],
        out_specs=[
            pl.BlockSpec((1, NP3, RED), lambda p, b, mb: (b, mb, 0)),
            pl.BlockSpec((1, RED, NP3), lambda p, b, mb: (b, 0, mb)),
        ],
        out_shape=[
            jax.ShapeDtypeStruct((B, N, RED), jnp.float32),
            jax.ShapeDtypeStruct((B, RED, N), jnp.float32),
        ],
        scratch_shapes=[pltpu.VMEM((8, RED), jnp.float32)],
        compiler_params=_ARB3,
    )(G, Wg.T, bg.reshape(1, RED), gg.reshape(1, RED), beg.reshape(1, RED))


QPW = 16


def _sc_lap(x2cm, idxf):
    mesh = plsc.VectorSubcoreMesh(core_axis_name="c", subcore_axis_name="s")

    @functools.partial(
        pl.kernel, mesh=mesh,
        out_type=jax.ShapeDtypeStruct((B * N * RED,), jnp.float32),
        compiler_params=_SC_PARAMS,
        scratch_types=[
            pltpu.VMEM((N * KL,), jnp.int32),
            pltpu.VMEM((QPW * N,), jnp.float32),
            pltpu.VMEM((QPW * 128 * 16,), jnp.float32),
            pltpu.SemaphoreType.DMA,
        ],
    )
    def k(x2_hbm, idx_hbm, out_hbm, idx_v, fb_v, acc_v, sem):
        wid = lax.axis_index("s") * 2 + lax.axis_index("c")
        b = wid // 4
        qbase = (wid % 4) * QPW
        pltpu.sync_copy(idx_hbm.at[b], idx_v)
        for ql in range(QPW):
            pltpu.sync_copy(x2_hbm.at[b, qbase + ql],
                            fb_v.at[pl.ds(ql * N, N)])

        for ql in range(QPW):
            fboff = ql * N
            arow = ql * 128

            def body(a, _):
                s = a >> 2
                t = a & 3
                m0 = (s * 64 + t) * 16
                acc = plsc.load_gather(
                    fb_v, [idx_v[pl.ds(m0, 16)] + fboff])
                for kk in range(1, KL):
                    acc = acc + plsc.load_gather(
                        fb_v, [idx_v[pl.ds(m0 + kk * 64, 16)] + fboff])
                acc_v[pl.ds((arow + a) * 16, 16)] = acc
                return 0
            lax.fori_loop(0, 128, body, 0, unroll=2)

        out0 = (b * N + 32 * qbase) * RED
        pltpu.sync_copy(acc_v, out_hbm.at[pl.ds(out0, QPW * 128 * 16)])

    return k(x2cm, idxf)


NB5 = 1024
CNT5 = float(B * N)


def _k5_body(x2_ref, mn_ref, wlt_ref, bl_ref, g_ref, be_ref, out_ref, st_ref):
    p = pl.program_id(0)
    b = pl.program_id(1)
    nb = pl.program_id(2)
    x2 = x2_ref[0]
    lap = x2 - mn_ref[0] * (1.0 / KL)
    traw = jax.lax.dot_general(
        lap, wlt_ref[...], (((1,), (0,)), ((), ())),
        preferred_element_type=jnp.float32) + bl_ref[...]

    @pl.when((p == 0) & (b == 0) & (nb == 0))
    def _():
        st_ref[...] = jnp.zeros_like(st_ref)

    @pl.when(p == 0)
    def _():
        st_ref[0:1, :] += jnp.sum(traw, axis=0, keepdims=True)
        st_ref[1:2, :] += jnp.sum(traw * traw, axis=0, keepdims=True)

    @pl.when(p == 1)
    def _():
        mean = st_ref[0:1, :] / CNT5
        var = st_ref[1:2, :] / CNT5 - mean * mean
        scale = g_ref[...] * jax.lax.rsqrt(var + EPS)
        shift = be_ref[...] - mean * scale
        x3 = x2 + jnp.maximum(traw * scale + shift, 0.0)
        out_ref[0] = x3.T


def _stage5(x2pm, meanS, Wl, bl, gl, bel):
    return pl.pallas_call(
        _k5_body,
        grid=(2, B, N // NB5),
        in_specs=[
            pl.BlockSpec((1, NB5, RED), lambda p, b, nb: (b, nb, 0)),
            pl.BlockSpec((1, NB5, RED), lambda p, b, nb: (b, nb, 0)),
            pl.BlockSpec((RED, RED), lambda p, b, nb: (0, 0)),
            pl.BlockSpec((1, RED), lambda p, b, nb: (0, 0)),
            pl.BlockSpec((1, RED), lambda p, b, nb: (0, 0)),
            pl.BlockSpec((1, RED), lambda p, b, nb: (0, 0)),
        ],
        out_specs=pl.BlockSpec((1, RED, NB5), lambda p, b, nb: (b, 0, nb)),
        out_shape=jax.ShapeDtypeStruct((B, RED, N), jnp.float32),
        scratch_shapes=[pltpu.VMEM((8, RED), jnp.float32)],
        compiler_params=_ARB3,
    )(x2pm, meanS, Wl.T, bl.reshape(1, RED), gl.reshape(1, RED),
      bel.reshape(1, RED))


NB6 = 1024
CNT6 = float(B * N)


def _k6_body(x3_ref, feat_ref, w2_ref, b2_ref, g2_ref, be2_ref,
             w3_ref, b3_ref, g3_ref, be3_ref, out_ref, st2_ref, st3_ref):
    p = pl.program_id(0)
    b = pl.program_id(1)
    nb = pl.program_id(2)
    yraw = jax.lax.dot_general(
        w2_ref[...], x3_ref[0], (((1,), (0,)), ((), ())),
        preferred_element_type=jnp.float32) + b2_ref[...]

    @pl.when((p == 0) & (b == 0) & (nb == 0))
    def _():
        st2_ref[...] = jnp.zeros_like(st2_ref)
        st3_ref[...] = jnp.zeros_like(st3_ref)

    @pl.when(p == 0)
    def _():
        st2_ref[:, 0:1] += jnp.sum(yraw, axis=1, keepdims=True)
        st2_ref[:, 1:2] += jnp.sum(yraw * yraw, axis=1, keepdims=True)

    @pl.when(p >= 1)
    def _():
        mean2 = st2_ref[:, 0:1] / CNT6
        var2 = st2_ref[:, 1:2] / CNT6 - mean2 * mean2
        scale2 = g2_ref[...] * jax.lax.rsqrt(var2 + EPS)
        shift2 = be2_ref[...] - mean2 * scale2
        y = jnp.maximum(yraw * scale2 + shift2, 0.0) + feat_ref[0]
        zraw = jax.lax.dot_general(
            w3_ref[...], y, (((1,), (0,)), ((), ())),
            preferred_element_type=jnp.float32) + b3_ref[...]

        @pl.when(p == 1)
        def _():
            st3_ref[:, 0:1] += jnp.sum(zraw, axis=1, keepdims=True)
            st3_ref[:, 1:2] += jnp.sum(zraw * zraw, axis=1, keepdims=True)

        @pl.when(p == 2)
        def _():
            mean3 = st3_ref[:, 0:1] / CNT6
            var3 = st3_ref[:, 1:2] / CNT6 - mean3 * mean3
            scale3 = g3_ref[...] * jax.lax.rsqrt(var3 + EPS)
            shift3 = be3_ref[...] - mean3 * scale3
            out_ref[0] = jnp.maximum(zraw * scale3 + shift3, 0.0)


def _stage6(x3cm, feat, W2, b2, g2, be2, W3, b3, g3, be3):
    C2 = 2 * CIN
    return pl.pallas_call(
        _k6_body,
        grid=(3, B, N // NB6),
        in_specs=[
            pl.BlockSpec((1, RED, NB6), lambda p, b, nb: (b, 0, nb)),
            pl.BlockSpec((1, CIN, NB6), lambda p, b, nb: (b, 0, nb)),
            pl.BlockSpec((CIN, RED), lambda p, b, nb: (0, 0)),
            pl.BlockSpec((CIN, 1), lambda p, b, nb: (0, 0)),
            pl.BlockSpec((CIN, 1), lambda p, b, nb: (0, 0)),
            pl.BlockSpec((CIN, 1), lambda p, b, nb: (0, 0)),
            pl.BlockSpec((C2, CIN), lambda p, b, nb: (0, 0)),
            pl.BlockSpec((C2, 1), lambda p, b, nb: (0, 0)),
            pl.BlockSpec((C2, 1), lambda p, b, nb: (0, 0)),
            pl.BlockSpec((C2, 1), lambda p, b, nb: (0, 0)),
        ],
        out_specs=pl.BlockSpec((1, C2, NB6), lambda p, b, nb: (b, 0, nb)),
        out_shape=jax.ShapeDtypeStruct((B, C2, N), jnp.float32),
        scratch_shapes=[
            pltpu.VMEM((CIN, 8), jnp.float32),
            pltpu.VMEM((C2, 8), jnp.float32),
        ],
        compiler_params=_ARB3,
    )(x3cm, feat, W2, b2.reshape(CIN, 1), g2.reshape(CIN, 1),
      be2.reshape(CIN, 1), W3, b3.reshape(C2, 1), g3.reshape(C2, 1),
      be3.reshape(C2, 1))


def kernel(xyz, feat, W1, b1, g1, be1, Wg, bg, gg, beg, Wl, bl, gl, bel,
           W2, b2, g2, be2, W3, b3, g3, be3):
    idx16 = _topk16(xyz)

    idx8 = idx16[:, :, :KG]
    Jg = jnp.transpose(idx8.reshape(B, KG, N // KG, KG), (0, 2, 3, 1))
    Jg = Jg.reshape(B, N * KG) + (jnp.arange(B, dtype=jnp.int32) * N)[:, None]
    Jg = Jg.reshape(B * N * KG // 128, 128)

    featT = jnp.transpose(feat, (0, 2, 1))
    z1 = _stage1(featT, W1, g1, be1)

    G = _sc_group_gather(z1.reshape(B * N, CIN), Jg)
    x2pm, x2cm = _stage3(G, Wg, bg, gg, beg)

    accf = _sc_lap(x2cm, idx16.reshape(B, N * KL))
    meanS = accf.reshape(B, N, RED)

    x3cm = _stage5(x2pm, meanS, Wl, bl, gl, bel)
    return _stage6(x3cm, feat, W2, b2, g2, be2, W3, b3, g3, be3)
